# Initial kernel scaffold; baseline (speedup 1.0000x reference)
import jax
import jax.numpy as jnp
from jax.experimental import pallas as pl


def kernel(indices, table):
    raise NotImplementedError("write your pallas kernel here")


# speedup vs baseline: 3.4662x; 3.4662x over previous
import functools

import jax
import jax.numpy as jnp
from jax import lax
from jax.experimental import pallas as pl
from jax.experimental.pallas import tpu as pltpu
from jax.experimental.pallas import tpu_sc as plsc

VOCAB = 100000
EMBED = 128
BATCH = 4096
HIST = 50

NC = 2
NS = 16
NW = NC * NS

TOTAL = BATCH * HIST
PER_W = TOTAL // NW
CHUNK = 128
N_CHUNKS = PER_W // CHUNK
NBUF = 5
N_GROUPS = N_CHUNKS // NBUF


def _gather_body(idx_hbm, table_hbm, out_hbm, idx_v, rows_v, *sems):
    gsem = sems[:NBUF]
    osem = sems[NBUF:]
    wid = lax.axis_index("s") * NC + lax.axis_index("c")
    row_base = wid * PER_W

    pltpu.sync_copy(idx_hbm.at[wid], idx_v)

    def start_gather(g, b):
        pltpu.async_copy(table_hbm.at[idx_v.at[g]], rows_v.at[b], gsem[b])

    def wait_gather(g, b):
        pltpu.make_async_copy(
            table_hbm.at[idx_v.at[g]], rows_v.at[b], gsem[b]
        ).wait()

    def start_out(g, b):
        pltpu.async_copy(
            rows_v.at[b], out_hbm.at[pl.ds(row_base + g * CHUNK, CHUNK)],
            osem[b],
        )

    def wait_out(g, b):
        pltpu.make_async_copy(
            rows_v.at[b], out_hbm.at[pl.ds(row_base + g * CHUNK, CHUNK)],
            osem[b],
        ).wait()

    for b in range(NBUF):
        start_gather(b, b)

    def group(t, _):
        for b in range(NBUF):
            g = t * NBUF + b
            wait_gather(g, b)
            start_out(g, b)
            wait_out(g, b)
            start_gather(g + NBUF, b)
        return 0

    lax.fori_loop(0, N_GROUPS - 1, group, 0)

    for b in range(NBUF):
        g = (N_GROUPS - 1) * NBUF + b
        wait_gather(g, b)
        start_out(g, b)
    for b in range(NBUF):
        g = (N_GROUPS - 1) * NBUF + b
        wait_out(g, b)


@jax.jit
def _embedding_gather(idx2d, table):
    mesh = plsc.VectorSubcoreMesh(
        core_axis_name="c", subcore_axis_name="s",
        num_cores=NC, num_subcores=NS,
    )
    k = pl.kernel(
        _gather_body,
        out_type=jax.ShapeDtypeStruct((TOTAL, EMBED), jnp.float32),
        mesh=mesh,
        scratch_types=[
            pltpu.VMEM((N_CHUNKS, CHUNK), jnp.int32),
            pltpu.VMEM((NBUF, CHUNK, EMBED), jnp.float32),
        ] + [pltpu.SemaphoreType.DMA] * (2 * NBUF),
    )
    return k(idx2d, table)


def kernel(indices, table):
    idx3d = indices.astype(jnp.int32).reshape(NW, N_CHUNKS, CHUNK)
    out = _embedding_gather(idx3d, table)
    return out.reshape(BATCH, HIST, EMBED)
